# baseline (speedup 1.0000x reference)
import functools

import jax
import jax.numpy as jnp
from jax import lax
from jax.experimental import pallas as pl
from jax.experimental.pallas import tpu as pltpu
from jax.experimental.pallas import tpu_sc as plsc

B = 4096
L = 200
HID = 64
NROWS = B * L

NC = 2
NS = 16
NW = NC * NS

ROWS_W = NROWS // NW
CH = L
NCH = ROWS_W // CH
M = 40
SUB = CH // M
NBUF = 4
IDXR_W = ROWS_W // M

_mesh = plsc.VectorSubcoreMesh(core_axis_name="c", subcore_axis_name="s")


@functools.partial(
    pl.kernel,
    mesh=_mesh,
    compiler_params=pltpu.CompilerParams(use_tc_tiling_on_sc=False),
    out_type=jax.ShapeDtypeStruct((NROWS, HID), jnp.float32),
    scratch_types=[
        pltpu.VMEM((IDXR_W, M), jnp.int32),
        pltpu.VMEM((L, HID), jnp.float32),
        pltpu.VMEM((CH, HID), jnp.float32),
        pltpu.VMEM((CH, HID), jnp.float32),
        pltpu.VMEM((CH, HID), jnp.float32),
        pltpu.VMEM((CH, HID), jnp.float32),
        pltpu.SemaphoreType.DMA,
        pltpu.SemaphoreType.DMA,
        pltpu.SemaphoreType.DMA,
        pltpu.SemaphoreType.DMA,
        pltpu.SemaphoreType.DMA,
        pltpu.SemaphoreType.DMA,
        pltpu.SemaphoreType.DMA,
        pltpu.SemaphoreType.DMA,
    ],
)
def _emb_kernel(x_hbm, tok_hbm, pos_hbm, out_hbm,
                idx_v, pos_v,
                buf0, buf1, buf2, buf3,
                sg0, sg1, sg2, sg3,
                ss0, ss1, ss2, ss3):
    bufs = (buf0, buf1, buf2, buf3)
    sgs = (sg0, sg1, sg2, sg3)
    sss = (ss0, ss1, ss2, ss3)

    wid = lax.axis_index("s") * NC + lax.axis_index("c")
    idx_row0 = wid * IDXR_W
    out_row0 = wid * ROWS_W

    pltpu.sync_copy(x_hbm.at[pl.ds(idx_row0, IDXR_W)], idx_v)
    pltpu.sync_copy(pos_hbm.at[pl.ds(0, L)], pos_v)

    def issue_gather(g, b):
        for j in range(SUB):
            pltpu.async_copy(
                tok_hbm.at[idx_v.at[g * SUB + j]],
                bufs[b].at[pl.ds(j * M, M)],
                sgs[b],
            )

    def wait_gather(b):
        pltpu.make_async_copy(
            out_hbm.at[pl.ds(0, CH)], bufs[b], sgs[b]
        ).wait()

    def issue_store(g, b):
        pltpu.async_copy(
            bufs[b], out_hbm.at[pl.ds(out_row0 + g * CH, CH)], sss[b]
        )

    def wait_store(b):
        pltpu.make_async_copy(
            bufs[b], out_hbm.at[pl.ds(0, CH)], sss[b]
        ).wait()

    def add_pos(b):
        buf = bufs[b]

        def body(i, carry):
            l0 = i * 4
            for r in range(4):
                for k in range(HID // 16):
                    sl = pl.ds(k * 16, 16)
                    plsc.addupdate(buf.at[l0 + r, sl], pos_v[l0 + r, sl])
            return carry

        lax.fori_loop(0, CH // 4, body, 0)

    for b in range(NBUF - 1):
        issue_gather(b, b)

    def chunk_iter(t, carry):
        for b in range(NBUF):
            g = t * NBUF + b
            wait_gather(b)
            add_pos(b)
            issue_store(g, b)
            nb = (b + NBUF - 1) % NBUF

            if b == 0:
                @pl.when(t > 0)
                def _():
                    wait_store(nb)

                issue_gather(g + NBUF - 1, nb)
            else:
                @pl.when(t < NCH // NBUF - 1)
                def _():
                    wait_store(nb)
                    issue_gather(g + NBUF - 1, nb)
        return carry

    lax.fori_loop(0, NCH // NBUF, chunk_iter, 0)

    for b in range(NBUF):
        wait_store(b)


def kernel(x, token_table, pos_table):
    x2d = x.astype(jnp.int32).reshape(NROWS // M, M)
    out = _emb_kernel(x2d, token_table, pos_table)
    return out.reshape(B, L, HID)
